# baseline (speedup 1.0000x reference)
import jax
import jax.numpy as jnp
from jax.experimental import pallas as pl
from jax.experimental.pallas import tpu as pltpu

_B, _C, _M = 256, 64, 4096
_KB = 8


def _body(x_ref, o_ref, c_ref, w_ref):
    K, P, M = x_ref.shape
    P2 = 2 * P
    xb = x_ref[...]
    mu = jnp.mean(xb, axis=2, keepdims=True)
    xc = (xb - mu).astype(jnp.bfloat16)

    covs = []
    for k in range(K):
        xck = xc[k]
        c = jax.lax.dot_general(
            xck, xck, (((1,), (1,)), ((), ())),
            preferred_element_type=jnp.float32)
        covs.append(c[:, None, :] * (1.0 / M))
    cov = jnp.concatenate(covs, axis=1)

    r0 = jax.lax.broadcasted_iota(jnp.int32, (P, 1, P), 0)
    l2 = jax.lax.broadcasted_iota(jnp.int32, (P, 1, P), 2)
    diagm = r0 == l2

    tr = jnp.sum(jnp.where(diagm, cov, 0.0), axis=(0, 2), keepdims=True)
    t2 = jnp.sum(cov * cov, axis=(0, 2), keepdims=True)
    n = float(M)
    num = (n - 2.0) / n * t2 + tr * tr
    den = (n + 2.0) * (t2 - tr * tr / P)
    rho = jnp.minimum(num / den, 1.0)
    A = (1.0 - rho) * cov + jnp.where(diagm, rho * tr * (1.0 / P), 0.0)

    ident = jnp.where(diagm, 1.0, 0.0)
    Bi = jnp.broadcast_to(ident, (P, K, P))
    C0 = jnp.concatenate([A, Bi], axis=2)
    c_ref[...] = C0

    lrow = jax.lax.broadcasted_iota(jnp.int32, (1, 1, P2), 2)
    r0full = jax.lax.broadcasted_iota(jnp.int32, (P, 1, 1), 0)
    diag0 = jnp.sum(jnp.where(diagm, A, 0.0), axis=0, keepdims=True)
    diag = jnp.concatenate(
        [diag0, jnp.zeros((1, K, P), jnp.float32)], axis=2)
    d0 = jnp.sum(jnp.where(lrow == P - 1, diag, 0.0), axis=2, keepdims=True)
    rinv0 = jax.lax.rsqrt(d0)
    dinv0 = rinv0 * rinv0
    row0 = jnp.sum(jnp.where(r0full == P - 1, C0, 0.0), axis=0,
                   keepdims=True)
    acol0 = jnp.sum(jnp.where(lrow == P - 1, C0, 0.0), axis=2,
                    keepdims=True)

    def step(i, carry):
        rinv, dinv, dg, row, acol = carry
        j = P - 1 - i
        jm = jnp.maximum(j - 1, 0)
        s = row * dinv
        w_ref[pl.ds(j, 1)] = row[:, :, P:] * rinv
        dg_n = dg - row * s
        d_n = jnp.sum(jnp.where(lrow == j - 1, dg_n, 0.0), axis=2,
                      keepdims=True)
        rinv_n = jax.lax.rsqrt(d_n)
        dinv_n = rinv_n * rinv_n
        row_old = c_ref[pl.ds(jm, 1)]
        a_corr = jnp.sum(jnp.where(lrow == j - 1, row, 0.0), axis=2,
                         keepdims=True)
        row_n = row_old - a_corr * s
        C = c_ref[...]
        colred = jnp.sum(jnp.where(lrow == j - 1, C, 0.0), axis=2,
                         keepdims=True)
        c_ref[...] = C - acol * s
        acol_n = colred - acol * (a_corr * dinv)
        return rinv_n, dinv_n, dg_n, row_n, acol_n

    jax.lax.fori_loop(0, P, step, (rinv0, dinv0, diag, row0, acol0),
                      unroll=32)

    WB = w_ref[...]
    Wt = jnp.swapaxes(WB, 0, 1).astype(jnp.bfloat16)
    for k in range(K):
        o_ref[k] = jnp.dot(Wt[k], xc[k],
                           preferred_element_type=jnp.float32)


def kernel(x):
    B, C, M = x.shape
    grid = (B // _KB,)
    return pl.pallas_call(
        _body,
        grid=grid,
        in_specs=[pl.BlockSpec((_KB, C, M), lambda i: (i, 0, 0))],
        out_specs=pl.BlockSpec((_KB, C, M), lambda i: (i, 0, 0)),
        out_shape=jax.ShapeDtypeStruct((B, C, M), jnp.float32),
        scratch_shapes=[pltpu.VMEM((C, _KB, 2 * C), jnp.float32),
                        pltpu.VMEM((C, _KB, C), jnp.float32)],
        compiler_params=pltpu.CompilerParams(
            dimension_semantics=("parallel",),
            vmem_limit_bytes=100 * 1024 * 1024,
        ),
    )(x)


# speedup vs baseline: 1.0730x; 1.0730x over previous
import jax
import jax.numpy as jnp
from jax.experimental import pallas as pl
from jax.experimental.pallas import tpu as pltpu

_B, _C, _M = 256, 64, 4096
_KB = 8


def _body(x_ref, o_ref, c_ref, w_ref):
    K, P, M = x_ref.shape
    P2 = 2 * P
    xb = x_ref[...]
    mu = jnp.mean(xb, axis=2, keepdims=True)
    xc = (xb - mu).astype(jnp.bfloat16)

    covs = []
    for k in range(K):
        xck = xc[k]
        c = jax.lax.dot_general(
            xck, xck, (((1,), (1,)), ((), ())),
            preferred_element_type=jnp.float32)
        covs.append(c[:, None, :] * (1.0 / M))
    cov = jnp.concatenate(covs, axis=1)

    r0 = jax.lax.broadcasted_iota(jnp.int32, (P, 1, P), 0)
    l2 = jax.lax.broadcasted_iota(jnp.int32, (P, 1, P), 2)
    diagm = r0 == l2

    tr = jnp.sum(jnp.where(diagm, cov, 0.0), axis=(0, 2), keepdims=True)
    t2 = jnp.sum(cov * cov, axis=(0, 2), keepdims=True)
    n = float(M)
    num = (n - 2.0) / n * t2 + tr * tr
    den = (n + 2.0) * (t2 - tr * tr / P)
    rho = jnp.minimum(num / den, 1.0)
    A = (1.0 - rho) * cov + jnp.where(diagm, rho * tr * (1.0 / P), 0.0)

    ident = jnp.where(diagm, 1.0, 0.0)
    Bi = jnp.broadcast_to(ident, (P, K, P))
    C0 = jnp.concatenate([A, Bi], axis=2)
    c_ref[...] = C0

    lrow = jax.lax.broadcasted_iota(jnp.int32, (1, 1, P2), 2)
    r0full = jax.lax.broadcasted_iota(jnp.int32, (P, 1, 1), 0)
    diag0 = jnp.sum(jnp.where(diagm, A, 0.0), axis=0, keepdims=True)
    diag = jnp.concatenate(
        [diag0, jnp.zeros((1, K, P), jnp.float32)], axis=2)
    d0 = jnp.sum(jnp.where(lrow == P - 1, diag, 0.0), axis=2, keepdims=True)
    rinv0 = jax.lax.rsqrt(d0)
    dinv0 = rinv0 * rinv0
    row0 = jnp.sum(jnp.where(r0full == P - 1, C0, 0.0), axis=0,
                   keepdims=True)
    acol0 = jnp.sum(jnp.where(lrow == P - 1, C0, 0.0), axis=2,
                    keepdims=True)

    def step(i, carry):
        rinv, dinv, dg, row, acol = carry
        j = P - 1 - i
        jm = jnp.maximum(j - 1, 0)
        s = row * dinv
        w_ref[pl.ds(j, 1)] = row[:, :, P:] * rinv
        dg_n = dg - row * s
        d_n = jnp.sum(jnp.where(lrow == j - 1, dg_n, 0.0), axis=2,
                      keepdims=True)
        rinv_n = jax.lax.rsqrt(d_n)
        dinv_n = rinv_n * rinv_n
        row_old = c_ref[pl.ds(jm, 1)]
        a_corr = jnp.sum(jnp.where(lrow == j - 1, row, 0.0), axis=2,
                         keepdims=True)
        row_n = row_old - a_corr * s
        C = c_ref[...]
        colred = jnp.sum(jnp.where(lrow == j - 1, C, 0.0), axis=2,
                         keepdims=True)
        c_ref[...] = C - acol * s
        acol_n = colred - acol * (a_corr * dinv)
        return rinv_n, dinv_n, dg_n, row_n, acol_n

    jax.lax.fori_loop(0, P, step, (rinv0, dinv0, diag, row0, acol0),
                      unroll=16)

    WB = w_ref[...]
    Wt = jnp.swapaxes(WB, 0, 1).astype(jnp.bfloat16)
    for k in range(K):
        o_ref[k] = jnp.dot(Wt[k], xc[k],
                           preferred_element_type=jnp.float32)


def kernel(x):
    B, C, M = x.shape
    grid = (B // _KB,)
    return pl.pallas_call(
        _body,
        grid=grid,
        in_specs=[pl.BlockSpec((_KB, C, M), lambda i: (i, 0, 0))],
        out_specs=pl.BlockSpec((_KB, C, M), lambda i: (i, 0, 0)),
        out_shape=jax.ShapeDtypeStruct((B, C, M), jnp.float32),
        scratch_shapes=[pltpu.VMEM((C, _KB, 2 * C), jnp.float32),
                        pltpu.VMEM((C, _KB, C), jnp.float32)],
        compiler_params=pltpu.CompilerParams(
            dimension_semantics=("parallel",),
            vmem_limit_bytes=100 * 1024 * 1024,
        ),
    )(x)
